# baseline (speedup 1.0000x reference)
import jax
import jax.numpy as jnp
from jax.experimental import pallas as pl
from jax.experimental.pallas import tpu as pltpu

B, S, H, R = 4, 2048, 1024, 4
TILE = 512


def _fused_kernel(x_ref, wm_ref, bm_ref, g_ref, b_ref, wg_ref, we_ref,
                  be_ref, wo_ref, bo_ref, out_ref):
    x = x_ref[...]
    h = x + jnp.dot(x, wm_ref[...], preferred_element_type=jnp.float32,
                    precision=jax.lax.Precision.HIGHEST) + bm_ref[...]
    mu = jnp.mean(h, axis=-1, keepdims=True)
    var = jnp.mean((h - mu) ** 2, axis=-1, keepdims=True)
    h = (h - mu) * jax.lax.rsqrt(var + 1e-5) * g_ref[...] + b_ref[...]

    logits = jnp.dot(h, wg_ref[...], preferred_element_type=jnp.float32,
                     precision=jax.lax.Precision.HIGHEST)
    m = jnp.max(logits, axis=-1, keepdims=True)
    e = jnp.exp(logits - m)
    p = e / jnp.sum(e, axis=-1, keepdims=True)

    acc = jnp.dot(p, be_ref[...], preferred_element_type=jnp.float32)
    for r in range(R):
        acc += p[:, r:r + 1] * jnp.dot(
            h, we_ref[r], preferred_element_type=jnp.float32,
            precision=jax.lax.Precision.HIGHEST)
    out_ref[...] = jnp.dot(
        acc, wo_ref[...], preferred_element_type=jnp.float32,
        precision=jax.lax.Precision.HIGHEST) + bo_ref[...]


def kernel(x, Wm, bm, gamma, beta, Wg, We, be, Wo, bo):
    xf = x.reshape(B * S, H)
    n_tiles = (B * S) // TILE
    full = lambda *shape: pl.BlockSpec(shape, lambda i: (0,) * len(shape))
    out = pl.pallas_call(
        _fused_kernel,
        grid=(n_tiles,),
        in_specs=[
            pl.BlockSpec((TILE, H), lambda i: (i, 0)),
            full(H, H),
            full(1, H),
            full(1, H),
            full(1, H),
            full(H, R),
            full(R, H, H),
            full(R, H),
            full(H, H),
            full(1, H),
        ],
        out_specs=pl.BlockSpec((TILE, H), lambda i: (i, 0)),
        out_shape=jax.ShapeDtypeStruct((B * S, H), jnp.float32),
    )(xf, Wm, bm.reshape(1, H), gamma.reshape(1, H), beta.reshape(1, H),
      Wg, We, be, Wo, bo.reshape(1, H))
    return out.reshape(B, S, H)


# speedup vs baseline: 4.7724x; 4.7724x over previous
import jax
import jax.numpy as jnp
from jax.experimental import pallas as pl
from jax.experimental.pallas import tpu as pltpu

B, S, H, R = 4, 2048, 1024, 4
TILE = 512


def _fused_kernel(x_ref, wm_ref, bm_ref, g_ref, b_ref, wg_ref, we_ref,
                  be_ref, wo_ref, bo_ref, out_ref):
    x = x_ref[...]
    h = x + jnp.dot(x.astype(jnp.bfloat16), wm_ref[...],
                    preferred_element_type=jnp.float32) + bm_ref[...]
    mu = jnp.mean(h, axis=-1, keepdims=True)
    var = jnp.mean((h - mu) ** 2, axis=-1, keepdims=True)
    h = (h - mu) * jax.lax.rsqrt(var + 1e-5) * g_ref[...] + b_ref[...]

    logits = jnp.dot(h, wg_ref[...], preferred_element_type=jnp.float32,
                     precision=jax.lax.Precision.HIGHEST)
    m = jnp.max(logits, axis=-1, keepdims=True)
    e = jnp.exp(logits - m)
    p = e / jnp.sum(e, axis=-1, keepdims=True)

    hb = h.astype(jnp.bfloat16)
    acc = jnp.dot(p, be_ref[...], preferred_element_type=jnp.float32)
    for r in range(R):
        acc += p[:, r:r + 1] * jnp.dot(
            hb, we_ref[r], preferred_element_type=jnp.float32)
    out_ref[...] = jnp.dot(
        acc.astype(jnp.bfloat16), wo_ref[...],
        preferred_element_type=jnp.float32) + bo_ref[...]


def kernel(x, Wm, bm, gamma, beta, Wg, We, be, Wo, bo):
    xf = x.reshape(B * S, H)
    n_tiles = (B * S) // TILE
    full = lambda *shape: pl.BlockSpec(shape, lambda i: (0,) * len(shape))
    out = pl.pallas_call(
        _fused_kernel,
        grid=(n_tiles,),
        in_specs=[
            pl.BlockSpec((TILE, H), lambda i: (i, 0)),
            full(H, H),
            full(1, H),
            full(1, H),
            full(1, H),
            full(H, R),
            full(R, H, H),
            full(R, H),
            full(H, H),
            full(1, H),
        ],
        out_specs=pl.BlockSpec((TILE, H), lambda i: (i, 0)),
        out_shape=jax.ShapeDtypeStruct((B * S, H), jnp.float32),
    )(xf, Wm.astype(jnp.bfloat16), bm.reshape(1, H), gamma.reshape(1, H),
      beta.reshape(1, H), Wg, We.astype(jnp.bfloat16), be,
      Wo.astype(jnp.bfloat16), bo.reshape(1, H))
    return out.reshape(B, S, H)


# speedup vs baseline: 5.8348x; 1.2226x over previous
import functools

import jax
import jax.numpy as jnp
from jax.experimental import pallas as pl
from jax.experimental.pallas import tpu as pltpu

B, S, H, R = 4, 2048, 1024, 4
TILE = 512
GPAD = 128


def _fused_kernel(x_ref, wm_ref, bm_ref, g_ref, b_ref, wg_ref, we_ref,
                  be_ref, wo_ref, bo_ref, out_ref, wide_ref, bias_ref):
    i = pl.program_id(0)

    @pl.when(i == 0)
    def _fold():
        wo = wo_ref[...]
        for r in range(R):
            wide_ref[:, r * H:(r + 1) * H] = jnp.dot(
                we_ref[r], wo, preferred_element_type=jnp.float32
            ).astype(jnp.bfloat16)
        wide_ref[:, R * H:] = wg_ref[...]
        bias_ref[...] = jnp.dot(be_ref[...], wo,
                                preferred_element_type=jnp.float32)

    x = x_ref[...]
    h = x + jnp.dot(x.astype(jnp.bfloat16), wm_ref[...],
                    preferred_element_type=jnp.float32) + bm_ref[...]
    mu = jnp.mean(h, axis=-1, keepdims=True)
    var = jnp.mean((h - mu) ** 2, axis=-1, keepdims=True)
    h = (h - mu) * jax.lax.rsqrt(var + 1e-5) * g_ref[...] + b_ref[...]

    wide = jnp.dot(h.astype(jnp.bfloat16), wide_ref[...],
                   preferred_element_type=jnp.float32)

    logits = wide[:, R * H:R * H + R]
    m = jnp.max(logits, axis=-1, keepdims=True)
    e = jnp.exp(logits - m)
    p = e / jnp.sum(e, axis=-1, keepdims=True)

    acc = jnp.dot(p, bias_ref[...], preferred_element_type=jnp.float32)
    for r in range(R):
        acc += p[:, r:r + 1] * wide[:, r * H:(r + 1) * H]
    out_ref[...] = acc + bo_ref[...]


def kernel(x, Wm, bm, gamma, beta, Wg, We, be, Wo, bo):
    xf = x.reshape(B * S, H)
    n_tiles = (B * S) // TILE
    wg_pad = jnp.zeros((H, GPAD), jnp.bfloat16).at[:, :R].set(
        Wg.astype(jnp.bfloat16))
    full = lambda *shape: pl.BlockSpec(shape, lambda i: (0,) * len(shape))
    out = pl.pallas_call(
        _fused_kernel,
        grid=(n_tiles,),
        in_specs=[
            pl.BlockSpec((TILE, H), lambda i: (i, 0)),
            full(H, H),
            full(1, H),
            full(1, H),
            full(1, H),
            full(H, GPAD),
            full(R, H, H),
            full(R, H),
            full(H, H),
            full(1, H),
        ],
        out_specs=pl.BlockSpec((TILE, H), lambda i: (i, 0)),
        out_shape=jax.ShapeDtypeStruct((B * S, H), jnp.float32),
        scratch_shapes=[
            pltpu.VMEM((H, R * H + GPAD), jnp.bfloat16),
            pltpu.VMEM((R, H), jnp.float32),
        ],
    )(xf, Wm.astype(jnp.bfloat16), bm.reshape(1, H), gamma.reshape(1, H),
      beta.reshape(1, H), wg_pad, We.astype(jnp.bfloat16),
      be.astype(jnp.bfloat16), Wo.astype(jnp.bfloat16), bo.reshape(1, H))
    return out.reshape(B, S, H)
